# baseline (speedup 1.0000x reference)
import jax
import jax.numpy as jnp
from jax.experimental import pallas as pl
from jax.experimental.pallas import tpu as pltpu

_CAPACITY = 65536
_EMBED = 256
_PTR = 0
_NFEAT = 4096

_CHUNK_ROWS = (2048, 2048, 4096, 8192, 12288, 12288, 12288, 12288)
_CHUNK_OFF = tuple(sum(_CHUNK_ROWS[:i]) for i in range(len(_CHUNK_ROWS)))
_NCH = len(_CHUNK_ROWS)
_MAXROWS = max(_CHUNK_ROWS)
_NBUF = 3
_SLACK = 1


def _body(f_hbm, m_hbm, o_hbm, fvmem, bufs, rowbuf,
          in_sems, out_sems, f_sem, row_sem):
    def in_copy(i):
        return pltpu.make_async_copy(
            m_hbm.at[pl.ds(_CHUNK_OFF[i], _CHUNK_ROWS[i]), :],
            bufs.at[i % _NBUF, pl.ds(0, _CHUNK_ROWS[i]), :],
            in_sems.at[i % _NBUF],
        )

    def out_copy(i):
        return pltpu.make_async_copy(
            bufs.at[i % _NBUF, pl.ds(0, _CHUNK_ROWS[i]), :],
            o_hbm.at[pl.ds(_CHUNK_OFF[i], _CHUNK_ROWS[i]), :],
            out_sems.at[i % _NBUF],
        )

    fcopy = pltpu.make_async_copy(f_hbm, fvmem, f_sem)
    fcopy.start()
    for b in range(_NBUF):
        in_copy(b).start()
    fcopy.wait()
    rowbuf[...] = jnp.sum(fvmem[...], axis=0, keepdims=True) * (1.0 / _NFEAT)

    patch = pltpu.make_async_copy(rowbuf, o_hbm.at[pl.ds(_PTR, 1), :], row_sem)
    for i in range(_NCH):
        in_copy(i).wait()
        out_copy(i).start()
        j = i - _SLACK
        if j >= 0:
            out_copy(j).wait()
            if j + _NBUF < _NCH:
                in_copy(j + _NBUF).start()
            if j == 0:
                patch.start()
    for j in range(max(0, _NCH - _SLACK), _NCH):
        out_copy(j).wait()
    patch.wait()


def kernel(feature, memory):
    return pl.pallas_call(
        _body,
        in_specs=[
            pl.BlockSpec(memory_space=pl.ANY),
            pl.BlockSpec(memory_space=pl.ANY),
        ],
        out_specs=pl.BlockSpec(memory_space=pl.ANY),
        out_shape=jax.ShapeDtypeStruct((_CAPACITY, _EMBED), jnp.float32),
        scratch_shapes=[
            pltpu.VMEM((_NFEAT, _EMBED), jnp.float32),
            pltpu.VMEM((_NBUF, _MAXROWS, _EMBED), jnp.float32),
            pltpu.VMEM((1, _EMBED), jnp.float32),
            pltpu.SemaphoreType.DMA((_NBUF,)),
            pltpu.SemaphoreType.DMA((_NBUF,)),
            pltpu.SemaphoreType.DMA,
            pltpu.SemaphoreType.DMA,
        ],
    )(feature, memory)


# speedup vs baseline: 1.0366x; 1.0366x over previous
import jax
import jax.numpy as jnp
from jax.experimental import pallas as pl
from jax.experimental.pallas import tpu as pltpu

_CAPACITY = 65536
_EMBED = 256
_PTR = 0
_NFEAT = 4096

_CHUNK_ROWS = (4096,) * 16
_CHUNK_OFF = tuple(sum(_CHUNK_ROWS[:i]) for i in range(len(_CHUNK_ROWS)))
_NCH = len(_CHUNK_ROWS)
_MAXROWS = max(_CHUNK_ROWS)
_NBUF = 10
_SLACK = 4


def _body(f_hbm, m_hbm, o_hbm, fvmem, bufs, rowbuf,
          in_sems, out_sems, f_sem, row_sem):
    def in_copy(i):
        return pltpu.make_async_copy(
            m_hbm.at[pl.ds(_CHUNK_OFF[i], _CHUNK_ROWS[i]), :],
            bufs.at[i % _NBUF, pl.ds(0, _CHUNK_ROWS[i]), :],
            in_sems.at[i % _NBUF],
        )

    def out_copy(i):
        return pltpu.make_async_copy(
            bufs.at[i % _NBUF, pl.ds(0, _CHUNK_ROWS[i]), :],
            o_hbm.at[pl.ds(_CHUNK_OFF[i], _CHUNK_ROWS[i]), :],
            out_sems.at[i % _NBUF],
        )

    fcopy = pltpu.make_async_copy(f_hbm, fvmem, f_sem)
    fcopy.start()
    for b in range(_NBUF):
        in_copy(b).start()
    fcopy.wait()
    rowbuf[...] = jnp.sum(fvmem[...], axis=0, keepdims=True) * (1.0 / _NFEAT)

    patch = pltpu.make_async_copy(rowbuf, o_hbm.at[pl.ds(_PTR, 1), :], row_sem)
    for i in range(_NCH):
        in_copy(i).wait()
        out_copy(i).start()
        j = i - _SLACK
        if j >= 0:
            out_copy(j).wait()
            if j + _NBUF < _NCH:
                in_copy(j + _NBUF).start()
            if j == 0:
                patch.start()
    for j in range(max(0, _NCH - _SLACK), _NCH):
        out_copy(j).wait()
    patch.wait()


def kernel(feature, memory):
    return pl.pallas_call(
        _body,
        in_specs=[
            pl.BlockSpec(memory_space=pl.ANY),
            pl.BlockSpec(memory_space=pl.ANY),
        ],
        out_specs=pl.BlockSpec(memory_space=pl.ANY),
        out_shape=jax.ShapeDtypeStruct((_CAPACITY, _EMBED), jnp.float32),
        scratch_shapes=[
            pltpu.VMEM((_NFEAT, _EMBED), jnp.float32),
            pltpu.VMEM((_NBUF, _MAXROWS, _EMBED), jnp.float32),
            pltpu.VMEM((1, _EMBED), jnp.float32),
            pltpu.SemaphoreType.DMA((_NBUF,)),
            pltpu.SemaphoreType.DMA((_NBUF,)),
            pltpu.SemaphoreType.DMA,
            pltpu.SemaphoreType.DMA,
        ],
    )(feature, memory)


# speedup vs baseline: 1.0480x; 1.0110x over previous
import jax
import jax.numpy as jnp
from jax.experimental import pallas as pl
from jax.experimental.pallas import tpu as pltpu

_CAPACITY = 65536
_EMBED = 256
_PTR = 0
_NFEAT = 4096

_CHUNK_ROWS = (16384,) * 4
_CHUNK_OFF = tuple(sum(_CHUNK_ROWS[:i]) for i in range(len(_CHUNK_ROWS)))
_NCH = len(_CHUNK_ROWS)
_MAXROWS = max(_CHUNK_ROWS)
_NBUF = 3
_SLACK = 1
_SPLIT = 2


def _body(f_hbm, m_hbm, o_hbm, fvmem, bufs, rowbuf,
          in_sems, out_sems, f_sem, row_sem):
    def in_copy(i, s):
        h = _CHUNK_ROWS[i] // _SPLIT
        return pltpu.make_async_copy(
            m_hbm.at[pl.ds(_CHUNK_OFF[i] + s * h, h), :],
            bufs.at[i % _NBUF, pl.ds(s * h, h), :],
            in_sems.at[i % _NBUF, s],
        )

    def out_copy(i, s):
        h = _CHUNK_ROWS[i] // _SPLIT
        return pltpu.make_async_copy(
            bufs.at[i % _NBUF, pl.ds(s * h, h), :],
            o_hbm.at[pl.ds(_CHUNK_OFF[i] + s * h, h), :],
            out_sems.at[i % _NBUF, s],
        )

    def start_in(i):
        for s in range(_SPLIT):
            in_copy(i, s).start()

    def wait_in(i):
        for s in range(_SPLIT):
            in_copy(i, s).wait()

    def start_out(i):
        for s in range(_SPLIT):
            out_copy(i, s).start()

    def wait_out(i):
        for s in range(_SPLIT):
            out_copy(i, s).wait()

    fcopy = pltpu.make_async_copy(f_hbm, fvmem, f_sem)
    fcopy.start()
    for b in range(_NBUF):
        start_in(b)
    fcopy.wait()
    rowbuf[...] = jnp.sum(fvmem[...], axis=0, keepdims=True) * (1.0 / _NFEAT)

    patch = pltpu.make_async_copy(rowbuf, o_hbm.at[pl.ds(_PTR, 1), :], row_sem)
    for i in range(_NCH):
        wait_in(i)
        start_out(i)
        j = i - _SLACK
        if j >= 0:
            wait_out(j)
            if j + _NBUF < _NCH:
                start_in(j + _NBUF)
            if j == 0:
                patch.start()
    for j in range(max(0, _NCH - _SLACK), _NCH):
        wait_out(j)
    patch.wait()


def kernel(feature, memory):
    return pl.pallas_call(
        _body,
        in_specs=[
            pl.BlockSpec(memory_space=pl.ANY),
            pl.BlockSpec(memory_space=pl.ANY),
        ],
        out_specs=pl.BlockSpec(memory_space=pl.ANY),
        out_shape=jax.ShapeDtypeStruct((_CAPACITY, _EMBED), jnp.float32),
        scratch_shapes=[
            pltpu.VMEM((_NFEAT, _EMBED), jnp.float32),
            pltpu.VMEM((_NBUF, _MAXROWS, _EMBED), jnp.float32),
            pltpu.VMEM((1, _EMBED), jnp.float32),
            pltpu.SemaphoreType.DMA((_NBUF, _SPLIT)),
            pltpu.SemaphoreType.DMA((_NBUF, _SPLIT)),
            pltpu.SemaphoreType.DMA,
            pltpu.SemaphoreType.DMA,
        ],
    )(feature, memory)
